# baseline (speedup 1.0000x reference)
import functools

import jax
import jax.numpy as jnp
from jax import lax
from jax.experimental import pallas as pl
from jax.experimental.pallas import tpu as pltpu
from jax.experimental.pallas import tpu_sc as plsc

N, D = 50000, 128
NC, NS = 2, 16
NW = NC * NS
CHUNK = 1568
R = 224
GROUPS = R // 16
NBLOCKS = CHUNK // R

_GATHER_DNUMS = lax.GatherDimensionNumbers(
    offset_dims=(), collapsed_slice_dims=(0,), start_index_map=(0,))


def _perm(x, idx):
    return lax.gather(
        x, idx.reshape(16, 1), _GATHER_DNUMS, (1,),
        mode=lax.GatherScatterMode.PROMISE_IN_BOUNDS)


def _body(gu_hbm, gi_hbm, out_hbm, ubuf, ibuf, obuf, sem_u, sem_i):
    wid = lax.axis_index("s") * NC + lax.axis_index("c")
    base_w = wid * CHUNK

    lanes = lax.iota(jnp.int32, 16)
    xor_idx = {h: lanes ^ h for h in (8, 4, 2, 1)}
    bitrev = (
        ((lanes & 1) << 3) | (((lanes >> 1) & 1) << 2)
        | (((lanes >> 2) & 1) << 1) | ((lanes >> 3) & 1))

    def combine(a, b, h):
        sel = (lanes & h) == 0
        return jnp.where(sel, a + _perm(a, xor_idx[h]), b + _perm(b, xor_idx[h]))

    def block(b, _):
        base = jnp.minimum(base_w + b * R, N - R)
        cu = pltpu.async_copy(gu_hbm.at[pl.ds(base, R), :], ubuf, sem_u)
        ci = pltpu.async_copy(gi_hbm.at[pl.ds(base, R), :], ibuf, sem_i)
        cu.wait()
        ci.wait()

        def group(g, _):
            r0 = g * 16
            vs = []
            for rr in range(16):
                r = r0 + rr
                ps = [ubuf[r, pl.ds(k * 16, 16)] * ibuf[r, pl.ds(k * 16, 16)]
                      for k in range(8)]
                while len(ps) > 1:
                    ps = [ps[i] + ps[i + 1] for i in range(0, len(ps), 2)]
                vs.append(ps[0])
            for h in (8, 4, 2, 1):
                vs = [combine(vs[i], vs[i + 1], h) for i in range(0, len(vs), 2)]
            obuf[pl.ds(r0, 16)] = _perm(vs[0], bitrev)
            return 0

        lax.fori_loop(0, GROUPS, group, 0)
        pltpu.sync_copy(obuf, out_hbm.at[pl.ds(base, R)])
        return 0

    lax.fori_loop(0, NBLOCKS, block, 0)


@jax.jit
def kernel(gu, gi):
    f = functools.partial(
        pl.kernel,
        mesh=plsc.VectorSubcoreMesh(core_axis_name="c", subcore_axis_name="s"),
        out_type=jax.ShapeDtypeStruct((N,), jnp.float32),
        scratch_types=[
            pltpu.VMEM((R, D), jnp.float32),
            pltpu.VMEM((R, D), jnp.float32),
            pltpu.VMEM((R,), jnp.float32),
            pltpu.SemaphoreType.DMA,
            pltpu.SemaphoreType.DMA,
        ],
    )(_body)
    return f(gu, gi)


# speedup vs baseline: 1.1721x; 1.1721x over previous
import functools

import jax
import jax.numpy as jnp
from jax import lax
from jax.experimental import pallas as pl
from jax.experimental.pallas import tpu as pltpu
from jax.experimental.pallas import tpu_sc as plsc

N, D = 50000, 128
NC, NS = 2, 16
NW = NC * NS
CHUNK = 1568
R = 112
GROUPS = R // 16
NBLOCKS = CHUNK // R
PAIRS = NBLOCKS // 2

_GATHER_DNUMS = lax.GatherDimensionNumbers(
    offset_dims=(), collapsed_slice_dims=(0,), start_index_map=(0,))


def _perm(x, idx):
    return lax.gather(
        x, idx.reshape(16, 1), _GATHER_DNUMS, (1,),
        mode=lax.GatherScatterMode.PROMISE_IN_BOUNDS)


def _body(gu_hbm, gi_hbm, out_hbm, ua, ia, ub, ib, obuf,
          sem_ua, sem_ia, sem_ub, sem_ib):
    wid = lax.axis_index("s") * NC + lax.axis_index("c")
    base_w = jnp.minimum(wid * CHUNK, N - CHUNK)

    lanes = lax.iota(jnp.int32, 16)
    xor_idx = {h: lanes ^ h for h in (8, 4, 2, 1)}
    bitrev = (
        ((lanes & 1) << 3) | (((lanes >> 1) & 1) << 2)
        | (((lanes >> 2) & 1) << 1) | ((lanes >> 3) & 1))

    def combine(a, b, h):
        sel = (lanes & h) == 0
        return jnp.where(sel, a + _perm(a, xor_idx[h]), b + _perm(b, xor_idx[h]))

    def start(base, u_ref, i_ref, su, si):
        pltpu.async_copy(gu_hbm.at[pl.ds(base, R), :], u_ref, su)
        pltpu.async_copy(gi_hbm.at[pl.ds(base, R), :], i_ref, si)

    def wait(u_ref, i_ref, su, si):
        pltpu.make_async_copy(gu_hbm.at[pl.ds(0, R), :], u_ref, su).wait()
        pltpu.make_async_copy(gi_hbm.at[pl.ds(0, R), :], i_ref, si).wait()

    def compute(u_ref, i_ref, off):
        def group(g, _):
            r0 = g * 16
            vs = []
            for rr in range(16):
                r = r0 + rr
                ps = [u_ref[r, pl.ds(k * 16, 16)] * i_ref[r, pl.ds(k * 16, 16)]
                      for k in range(8)]
                while len(ps) > 1:
                    ps = [ps[i] + ps[i + 1] for i in range(0, len(ps), 2)]
                vs.append(ps[0])
            acc = vs
            for h in (8, 4, 2, 1):
                acc = [combine(acc[i], acc[i + 1], h)
                       for i in range(0, len(acc), 2)]
            obuf[pl.ds(off + r0, 16)] = _perm(acc[0], bitrev)
            return 0

        lax.fori_loop(0, GROUPS, group, 0)

    start(base_w, ua, ia, sem_ua, sem_ia)

    def pair(p, _):
        b0 = 2 * p
        start(base_w + (b0 + 1) * R, ub, ib, sem_ub, sem_ib)
        wait(ua, ia, sem_ua, sem_ia)
        compute(ua, ia, b0 * R)

        @pl.when(p < PAIRS - 1)
        def _():
            start(base_w + (b0 + 2) * R, ua, ia, sem_ua, sem_ia)

        wait(ub, ib, sem_ub, sem_ib)
        compute(ub, ib, (b0 + 1) * R)
        return 0

    lax.fori_loop(0, PAIRS, pair, 0)
    pltpu.sync_copy(obuf, out_hbm.at[pl.ds(base_w, CHUNK)])


@jax.jit
def kernel(gu, gi):
    f = functools.partial(
        pl.kernel,
        mesh=plsc.VectorSubcoreMesh(core_axis_name="c", subcore_axis_name="s"),
        out_type=jax.ShapeDtypeStruct((N,), jnp.float32),
        scratch_types=[
            pltpu.VMEM((R, D), jnp.float32),
            pltpu.VMEM((R, D), jnp.float32),
            pltpu.VMEM((R, D), jnp.float32),
            pltpu.VMEM((R, D), jnp.float32),
            pltpu.VMEM((CHUNK,), jnp.float32),
            pltpu.SemaphoreType.DMA,
            pltpu.SemaphoreType.DMA,
            pltpu.SemaphoreType.DMA,
            pltpu.SemaphoreType.DMA,
        ],
    )(_body)
    return f(gu, gi)
